# baseline (speedup 1.0000x reference)
import jax
import jax.numpy as jnp
from jax import lax
from jax.experimental import pallas as pl
from jax.experimental.pallas import tpu as pltpu


def _diag_project_body(dim, x_ref, p_ref, o_ref):
    t = pl.program_id(0)
    blk = x_ref[...]
    r = lax.broadcasted_iota(jnp.int32, (128, 128), 0)
    c = lax.broadcasted_iota(jnp.int32, (128, 128), 1)
    eye = (r == c).astype(jnp.float32)
    diag = jnp.sum(blk * eye[None], axis=1)
    lane = lax.broadcasted_iota(jnp.int32, diag.shape, 1)
    diag = jnp.where(128 * t + lane < dim, diag, 0.0)
    contrib = jnp.dot(diag, p_ref[...], preferred_element_type=jnp.float32)

    @pl.when(t == 0)
    def _init():
        o_ref[...] = contrib

    @pl.when(t != 0)
    def _acc():
        o_ref[...] += contrib


def kernel(input, P):
    batch, dim, _ = input.shape
    s = P.shape[1]
    dpad = ((dim + 127) // 128) * 128
    nblk = dpad // 128

    p_pad = jnp.pad(P, ((0, dpad - dim), (0, 0)))
    return pl.pallas_call(
        lambda x, p, o: _diag_project_body(dim, x, p, o),
        grid=(nblk,),
        in_specs=[
            pl.BlockSpec((batch, 128, 128), lambda t: (0, t, t)),
            pl.BlockSpec((128, s), lambda t: (t, 0)),
        ],
        out_specs=pl.BlockSpec((batch, s), lambda t: (0, 0)),
        out_shape=jax.ShapeDtypeStruct((batch, s), jnp.float32),
    )(input, p_pad)


# speedup vs baseline: 15.3782x; 15.3782x over previous
import jax
import jax.numpy as jnp
from jax import lax
from jax.experimental import pallas as pl

_RB = 64


def _diag_project_body(dim, x_ref, p_ref, o_ref):
    i = pl.program_id(0)
    off = _RB * lax.rem(i, 128 // _RB)
    blk = x_ref[...]
    rr = lax.broadcasted_iota(jnp.int32, (_RB, 1, 128), 0)
    cc = lax.broadcasted_iota(jnp.int32, (_RB, 1, 128), 2)
    z = jnp.where(cc == rr + off, blk, 0.0)
    g = jnp.sum(z, axis=0)
    lane = lax.broadcasted_iota(jnp.int32, g.shape, 1)
    base = 128 * lax.div(i, 128 // _RB)
    g = jnp.where(base + lane < dim, g, 0.0)
    contrib = jnp.dot(g, p_ref[...], preferred_element_type=jnp.float32)

    @pl.when(i == 0)
    def _init():
        o_ref[...] = contrib

    @pl.when(i != 0)
    def _acc():
        o_ref[...] += contrib


def kernel(input, P):
    batch, dim, _ = input.shape
    s = P.shape[1]
    dpad = ((dim + 127) // 128) * 128
    nsub = 128 // _RB

    xt = jnp.transpose(input, (1, 0, 2))
    p_pad = jnp.pad(P, ((0, dpad - dim), (0, 0)))
    return pl.pallas_call(
        lambda x, p, o: _diag_project_body(dim, x, p, o),
        grid=(dpad // _RB,),
        in_specs=[
            pl.BlockSpec((_RB, batch, 128), lambda i: (i, 0, i // nsub)),
            pl.BlockSpec((128, s), lambda i: (i // nsub, 0)),
        ],
        out_specs=pl.BlockSpec((batch, s), lambda i: (0, 0)),
        out_shape=jax.ShapeDtypeStruct((batch, s), jnp.float32),
    )(xt, p_pad)
